# baseline (speedup 1.0000x reference)
import functools

import jax
import jax.numpy as jnp
from jax import lax
from jax.experimental import pallas as pl
from jax.experimental.pallas import tpu as pltpu
from jax.experimental.pallas import tpu_sc as plsc

N = 10000
D = 128
E = 320000

NC = 2
NS = 16
NW = NC * NS
EPW = E // NW
CH = 80
NCHUNK = EPW // CH
NB = 4
CHD = 80
NCHD = EPW // CHD
RSTG = 200
NSTG = N // RSTG

RB = 10000
GRID = N // RB


def _sc_degree(col):
    mesh = plsc.VectorSubcoreMesh(core_axis_name="c", subcore_axis_name="s")

    @functools.partial(
        pl.kernel,
        out_type=jax.ShapeDtypeStruct((NC * N,), jnp.float32),
        mesh=mesh,
        scratch_types=[
            pltpu.VMEM((2000,), jnp.float32),
            pltpu.VMEM((128,), jnp.float32),
            pltpu.VMEM((4, CHD), jnp.int32),
            pltpu.SemaphoreType.DMA,
            pltpu.VMEM((N,), jnp.float32),
            pltpu.VMEM_SHARED((N,), jnp.float32),
        ],
    )
    def deg_kernel(col_hbm, out_hbm, zbuf, ones_v, cbuf, csem, stage, acc):
        c = lax.axis_index("c")
        s = lax.axis_index("s")
        w = c * NS + s

        for i in range(8):
            ones_v[pl.ds(i * 16, 16)] = jnp.ones((16,), jnp.float32)

        @pl.when(s == 0)
        def _zero():
            def zb(i, carry):
                zbuf[pl.ds(i * 16, 16)] = jnp.zeros((16,), jnp.float32)
                return carry
            lax.fori_loop(0, 2000 // 16, zb, None)
            for j in range(N // 2000):
                pltpu.sync_copy(zbuf, acc.at[pl.ds(j * 2000, 2000)])

        plsc.subcore_barrier()

        for u in range(4):
            pltpu.async_copy(col_hbm.at[pl.ds(w * EPW + u * CHD, CHD)],
                             cbuf.at[u], csem)

        def body(i, carry):
            for u in range(4):
                k = 4 * i + u
                pltpu.make_async_copy(col_hbm.at[pl.ds(0, CHD)], cbuf.at[u],
                                      csem).wait()
                pltpu.sync_copy(ones_v.at[pl.ds(0, CHD)],
                                acc.at[cbuf.at[u]], add=True)

                @pl.when(k + 4 < NCHD)
                def _ld():
                    pltpu.async_copy(
                        col_hbm.at[pl.ds(w * EPW + (k + 4) * CHD, CHD)],
                        cbuf.at[u], csem)
            return carry
        lax.fori_loop(0, NCHD // 4, body, None)
        for u in range(NCHD % 4):
            pltpu.make_async_copy(col_hbm.at[pl.ds(0, CHD)], cbuf.at[u],
                                  csem).wait()
            pltpu.sync_copy(ones_v.at[pl.ds(0, CHD)], acc.at[cbuf.at[u]],
                            add=True)

        plsc.subcore_barrier()

        @pl.when(s == 0)
        def _writeback():
            pltpu.sync_copy(acc, stage)
            pltpu.sync_copy(stage, out_hbm.at[pl.ds(c * N, N)])

    return deg_kernel(col)


def _sc_scatter(h, row, col):
    mesh = plsc.VectorSubcoreMesh(core_axis_name="c", subcore_axis_name="s")

    @functools.partial(
        pl.kernel,
        out_type=jax.ShapeDtypeStruct((NC, N, D), jnp.float32),
        mesh=mesh,
        scratch_types=[
            pltpu.VMEM((NB * CH, D), jnp.float32),
            pltpu.VMEM((NB, CH), jnp.int32),
            pltpu.VMEM((NB, CH), jnp.int32),
            [pltpu.SemaphoreType.DMA] * NB,
            [pltpu.SemaphoreType.DMA] * NB,
            pltpu.SemaphoreType.DMA,
            pltpu.VMEM_SHARED((N, D), jnp.float32),
        ],
    )
    def scat_kernel(h_hbm, row_hbm, col_hbm, out_hbm,
                    gbuf, rbuf, cbuf, gsems, ssems, isem, acc):
        c = lax.axis_index("c")
        s = lax.axis_index("s")
        w = c * NS + s

        def zb(i, carry):
            for cc in range(D // 16):
                gbuf[i, pl.ds(cc * 16, 16)] = jnp.zeros((16,), jnp.float32)
            return carry
        lax.fori_loop(0, RSTG, zb, None)
        for k in range(NSTG // NS + 1):
            j = s + NS * k

            @pl.when(j < NSTG)
            def _zc():
                pltpu.sync_copy(gbuf.at[pl.ds(0, RSTG)],
                                acc.at[pl.ds(j * RSTG, RSTG)])

        plsc.subcore_barrier()

        def load_idx(it, slot):
            base = w * EPW + it * CH
            pltpu.async_copy(row_hbm.at[pl.ds(base, CH)], rbuf.at[slot],
                             isem)
            pltpu.async_copy(col_hbm.at[pl.ds(base, CH)], cbuf.at[slot],
                             isem)

        def wait_idx(slot):
            pltpu.make_async_copy(row_hbm.at[pl.ds(0, CH)], rbuf.at[slot],
                                  isem).wait()
            pltpu.make_async_copy(col_hbm.at[pl.ds(0, CH)], cbuf.at[slot],
                                  isem).wait()

        def start_gather(slot):
            pltpu.async_copy(h_hbm.at[rbuf.at[slot]],
                             gbuf.at[pl.ds(slot * CH, CH)], gsems[slot])

        def wait_gather(slot):
            pltpu.make_async_copy(h_hbm.at[rbuf.at[slot]],
                                  gbuf.at[pl.ds(slot * CH, CH)],
                                  gsems[slot]).wait()

        def start_scatter(slot):
            pltpu.async_copy(gbuf.at[pl.ds(slot * CH, CH)],
                             acc.at[cbuf.at[slot]], ssems[slot], add=True)

        def wait_scatter(slot):
            pltpu.make_async_copy(gbuf.at[pl.ds(slot * CH, CH)],
                                  acc.at[cbuf.at[slot]], ssems[slot]).wait()

        for u in range(NB):
            load_idx(u, u)
            wait_idx(u)
            start_gather(u)

        def ring(i, carry):
            for u in range(NB):
                k = NB * i + u
                wait_gather(u)
                start_scatter(u)
                up = (u + NB - 1) % NB
                r = k + NB - 1

                @pl.when(jnp.logical_and(r >= NB, r < NCHUNK))
                def _refill():
                    wait_scatter(up)
                    load_idx(r, up)
                    wait_idx(up)
                    start_gather(up)
            return carry
        lax.fori_loop(0, NCHUNK // NB, ring, None)

        for u in range(NCHUNK % NB):
            wait_gather(u)
            start_scatter(u)
        for u in range(NB):
            wait_scatter(u)

        plsc.subcore_barrier()

        for k in range(NSTG // NS + 1):
            j = s + NS * k

            @pl.when(j < NSTG)
            def _wb():
                pltpu.sync_copy(acc.at[pl.ds(j * RSTG, RSTG)],
                                out_hbm.at[c, pl.ds(j * RSTG, RSTG)])

    return scat_kernel(h, row, col)


def _tc_layer1(x, W0, W1, b2, degp):
    def body(x_ref, W0_ref, W1_ref, b_ref, deg_ref, out0_ref, h_ref,
             dinv_ref):
        deg = deg_ref[0] + deg_ref[1]
        dinv = jnp.where(deg > 0, lax.rsqrt(jnp.maximum(deg, 1e-12)), 0.0)
        xb = x_ref[...]
        out0_ref[...] = (
            jnp.dot(xb, W0_ref[...], preferred_element_type=jnp.float32)
            + b_ref[...]
        )
        h_ref[...] = dinv * jnp.dot(
            xb, W1_ref[...], preferred_element_type=jnp.float32)
        dinv_ref[...] = dinv

    return pl.pallas_call(
        body,
        grid=(GRID,),
        in_specs=[
            pl.BlockSpec((RB, D), lambda i: (i, 0)),
            pl.BlockSpec((D, D), lambda i: (0, 0)),
            pl.BlockSpec((D, D), lambda i: (0, 0)),
            pl.BlockSpec((1, D), lambda i: (0, 0)),
            pl.BlockSpec((NC, RB, 1), lambda i: (0, i, 0)),
        ],
        out_specs=[
            pl.BlockSpec((RB, D), lambda i: (i, 0)),
            pl.BlockSpec((RB, D), lambda i: (i, 0)),
            pl.BlockSpec((RB, 1), lambda i: (i, 0)),
        ],
        out_shape=[
            jax.ShapeDtypeStruct((N, D), jnp.float32),
            jax.ShapeDtypeStruct((N, D), jnp.float32),
            jax.ShapeDtypeStruct((N, 1), jnp.float32),
        ],
    )(x, W0, W1, b2, degp)


def _tc_layer2(out0, sp, dinv, W0, W1):
    def body(out0_ref, sp_ref, dinv_ref, W0_ref, W1_ref, out02_ref, h_ref):
        dinv = dinv_ref[...]
        t = out0_ref[...] + dinv * (sp_ref[0] + sp_ref[1])
        y = jnp.maximum(t, 0.01 * t)
        out02_ref[...] = jnp.dot(
            y, W0_ref[...], preferred_element_type=jnp.float32)
        h_ref[...] = dinv * jnp.dot(
            y, W1_ref[...], preferred_element_type=jnp.float32)

    return pl.pallas_call(
        body,
        grid=(GRID,),
        in_specs=[
            pl.BlockSpec((RB, D), lambda i: (i, 0)),
            pl.BlockSpec((NC, RB, D), lambda i: (0, i, 0)),
            pl.BlockSpec((RB, 1), lambda i: (i, 0)),
            pl.BlockSpec((D, D), lambda i: (0, 0)),
            pl.BlockSpec((D, D), lambda i: (0, 0)),
        ],
        out_specs=[
            pl.BlockSpec((RB, D), lambda i: (i, 0)),
            pl.BlockSpec((RB, D), lambda i: (i, 0)),
        ],
        out_shape=[
            jax.ShapeDtypeStruct((N, D), jnp.float32),
            jax.ShapeDtypeStruct((N, D), jnp.float32),
        ],
    )(out0, sp, dinv, W0, W1)


def _tc_final(out02, sp, dinv):
    def body(o_ref, sp_ref, dinv_ref, out_ref):
        t = o_ref[...] + dinv_ref[...] * (sp_ref[0] + sp_ref[1])
        out_ref[...] = jnp.maximum(t, 0.01 * t)

    return pl.pallas_call(
        body,
        grid=(GRID,),
        in_specs=[
            pl.BlockSpec((RB, D), lambda i: (i, 0)),
            pl.BlockSpec((NC, RB, D), lambda i: (0, i, 0)),
            pl.BlockSpec((RB, 1), lambda i: (i, 0)),
        ],
        out_specs=pl.BlockSpec((RB, D), lambda i: (i, 0)),
        out_shape=jax.ShapeDtypeStruct((N, D), jnp.float32),
    )(out02, sp, dinv)


def kernel(x, edge_index, W0_0, W1_0, b0, W0_1, W1_1):
    row = edge_index[0]
    col = edge_index[1]
    degp = _sc_degree(col).reshape(NC, N, 1)
    out0, h1, dinv = _tc_layer1(x, W0_0, W1_0, b0.reshape(1, D), degp)
    s1 = _sc_scatter(h1, row, col)
    out02, h2 = _tc_layer2(out0, s1, dinv, W0_1, W1_1)
    s2 = _sc_scatter(h2, row, col)
    return _tc_final(out02, s2, dinv)


# speedup vs baseline: 1.0138x; 1.0138x over previous
import functools

import jax
import jax.numpy as jnp
from jax import lax
from jax.experimental import pallas as pl
from jax.experimental.pallas import tpu as pltpu
from jax.experimental.pallas import tpu_sc as plsc

N = 10000
D = 128
E = 320000

NC = 2
NS = 16
NW = NC * NS
EPW = E // NW
CH = 80
NCHUNK = EPW // CH
NB = 4
CHD = 80
NCHD = EPW // CHD
RSTG = 200
NSTG = N // RSTG

RB = 5000
GRID = N // RB


def _sc_degree(col):
    mesh = plsc.VectorSubcoreMesh(core_axis_name="c", subcore_axis_name="s")

    @functools.partial(
        pl.kernel,
        out_type=jax.ShapeDtypeStruct((NC * N,), jnp.float32),
        mesh=mesh,
        scratch_types=[
            pltpu.VMEM((2000,), jnp.float32),
            pltpu.VMEM((128,), jnp.float32),
            pltpu.VMEM((4, CHD), jnp.int32),
            pltpu.SemaphoreType.DMA,
            pltpu.VMEM((N,), jnp.float32),
            pltpu.VMEM_SHARED((N,), jnp.float32),
        ],
    )
    def deg_kernel(col_hbm, out_hbm, zbuf, ones_v, cbuf, csem, stage, acc):
        c = lax.axis_index("c")
        s = lax.axis_index("s")
        w = c * NS + s

        for i in range(8):
            ones_v[pl.ds(i * 16, 16)] = jnp.ones((16,), jnp.float32)

        @pl.when(s == 0)
        def _zero():
            def zb(i, carry):
                zbuf[pl.ds(i * 16, 16)] = jnp.zeros((16,), jnp.float32)
                return carry
            lax.fori_loop(0, 2000 // 16, zb, None)
            for j in range(N // 2000):
                pltpu.sync_copy(zbuf, acc.at[pl.ds(j * 2000, 2000)])

        plsc.subcore_barrier()

        for u in range(4):
            pltpu.async_copy(col_hbm.at[pl.ds(w * EPW + u * CHD, CHD)],
                             cbuf.at[u], csem)

        def body(i, carry):
            for u in range(4):
                k = 4 * i + u
                pltpu.make_async_copy(col_hbm.at[pl.ds(0, CHD)], cbuf.at[u],
                                      csem).wait()
                pltpu.sync_copy(ones_v.at[pl.ds(0, CHD)],
                                acc.at[cbuf.at[u]], add=True)

                @pl.when(k + 4 < NCHD)
                def _ld():
                    pltpu.async_copy(
                        col_hbm.at[pl.ds(w * EPW + (k + 4) * CHD, CHD)],
                        cbuf.at[u], csem)
            return carry
        lax.fori_loop(0, NCHD // 4, body, None)
        for u in range(NCHD % 4):
            pltpu.make_async_copy(col_hbm.at[pl.ds(0, CHD)], cbuf.at[u],
                                  csem).wait()
            pltpu.sync_copy(ones_v.at[pl.ds(0, CHD)], acc.at[cbuf.at[u]],
                            add=True)

        plsc.subcore_barrier()

        @pl.when(s == 0)
        def _writeback():
            pltpu.sync_copy(acc, stage)
            pltpu.sync_copy(stage, out_hbm.at[pl.ds(c * N, N)])

    return deg_kernel(col)


def _sc_scatter(h, row, col):
    mesh = plsc.VectorSubcoreMesh(core_axis_name="c", subcore_axis_name="s")

    @functools.partial(
        pl.kernel,
        out_type=jax.ShapeDtypeStruct((NC, N, D), jnp.float32),
        mesh=mesh,
        scratch_types=[
            pltpu.VMEM((NB * CH, D), jnp.float32),
            pltpu.VMEM((NB, CH), jnp.int32),
            pltpu.VMEM((NB, CH), jnp.int32),
            [pltpu.SemaphoreType.DMA] * NB,
            [pltpu.SemaphoreType.DMA] * NB,
            pltpu.SemaphoreType.DMA,
            pltpu.VMEM_SHARED((N, D), jnp.float32),
        ],
    )
    def scat_kernel(h_hbm, row_hbm, col_hbm, out_hbm,
                    gbuf, rbuf, cbuf, gsems, ssems, isem, acc):
        c = lax.axis_index("c")
        s = lax.axis_index("s")
        w = c * NS + s

        def zb(i, carry):
            for cc in range(D // 16):
                gbuf[i, pl.ds(cc * 16, 16)] = jnp.zeros((16,), jnp.float32)
            return carry
        lax.fori_loop(0, RSTG, zb, None)
        for k in range(NSTG // NS + 1):
            j = s + NS * k

            @pl.when(j < NSTG)
            def _zc():
                pltpu.sync_copy(gbuf.at[pl.ds(0, RSTG)],
                                acc.at[pl.ds(j * RSTG, RSTG)])

        plsc.subcore_barrier()

        def load_idx(it, slot):
            base = w * EPW + it * CH
            pltpu.async_copy(row_hbm.at[pl.ds(base, CH)], rbuf.at[slot],
                             isem)
            pltpu.async_copy(col_hbm.at[pl.ds(base, CH)], cbuf.at[slot],
                             isem)

        def wait_idx(slot):
            pltpu.make_async_copy(row_hbm.at[pl.ds(0, CH)], rbuf.at[slot],
                                  isem).wait()
            pltpu.make_async_copy(col_hbm.at[pl.ds(0, CH)], cbuf.at[slot],
                                  isem).wait()

        def start_gather(slot):
            pltpu.async_copy(h_hbm.at[rbuf.at[slot]],
                             gbuf.at[pl.ds(slot * CH, CH)], gsems[slot])

        def wait_gather(slot):
            pltpu.make_async_copy(h_hbm.at[rbuf.at[slot]],
                                  gbuf.at[pl.ds(slot * CH, CH)],
                                  gsems[slot]).wait()

        def start_scatter(slot):
            pltpu.async_copy(gbuf.at[pl.ds(slot * CH, CH)],
                             acc.at[cbuf.at[slot]], ssems[slot], add=True)

        def wait_scatter(slot):
            pltpu.make_async_copy(gbuf.at[pl.ds(slot * CH, CH)],
                                  acc.at[cbuf.at[slot]], ssems[slot]).wait()

        for u in range(NB):
            load_idx(u, u)
            wait_idx(u)
            start_gather(u)

        def ring(i, carry):
            for u in range(NB):
                k = NB * i + u
                wait_gather(u)
                start_scatter(u)
                up = (u + NB - 1) % NB
                r = k + NB - 1

                @pl.when(jnp.logical_and(r >= NB, r < NCHUNK))
                def _refill():
                    wait_scatter(up)
                    load_idx(r, up)
                    wait_idx(up)
                    start_gather(up)
            return carry
        lax.fori_loop(0, NCHUNK // NB, ring, None)

        for u in range(NCHUNK % NB):
            wait_gather(u)
            start_scatter(u)
        for u in range(NB):
            wait_scatter(u)

        plsc.subcore_barrier()

        for k in range(NSTG // NS + 1):
            j = s + NS * k

            @pl.when(j < NSTG)
            def _wb():
                pltpu.sync_copy(acc.at[pl.ds(j * RSTG, RSTG)],
                                out_hbm.at[c, pl.ds(j * RSTG, RSTG)])

    return scat_kernel(h, row, col)


def _tc_layer1(x, W0, W1, b2, degp):
    def body(x_ref, W0_ref, W1_ref, b_ref, deg_ref, out0_ref, h_ref,
             dinv_ref):
        deg = deg_ref[0] + deg_ref[1]
        dinv = jnp.where(deg > 0, lax.rsqrt(jnp.maximum(deg, 1e-12)), 0.0)
        xb = x_ref[...]
        out0_ref[...] = (
            jnp.dot(xb, W0_ref[...], preferred_element_type=jnp.float32)
            + b_ref[...]
        )
        h_ref[...] = dinv * jnp.dot(
            xb, W1_ref[...], preferred_element_type=jnp.float32)
        dinv_ref[...] = dinv

    return pl.pallas_call(
        body,
        grid=(GRID,),
        in_specs=[
            pl.BlockSpec((RB, D), lambda i: (i, 0)),
            pl.BlockSpec((D, D), lambda i: (0, 0)),
            pl.BlockSpec((D, D), lambda i: (0, 0)),
            pl.BlockSpec((1, D), lambda i: (0, 0)),
            pl.BlockSpec((NC, RB, 1), lambda i: (0, i, 0)),
        ],
        out_specs=[
            pl.BlockSpec((RB, D), lambda i: (i, 0)),
            pl.BlockSpec((RB, D), lambda i: (i, 0)),
            pl.BlockSpec((RB, 1), lambda i: (i, 0)),
        ],
        out_shape=[
            jax.ShapeDtypeStruct((N, D), jnp.float32),
            jax.ShapeDtypeStruct((N, D), jnp.float32),
            jax.ShapeDtypeStruct((N, 1), jnp.float32),
        ],
    )(x, W0, W1, b2, degp)


def _tc_layer2(out0, sp, dinv, W0, W1):
    def body(out0_ref, sp_ref, dinv_ref, W0_ref, W1_ref, out02_ref, h_ref):
        dinv = dinv_ref[...]
        t = out0_ref[...] + dinv * (sp_ref[0] + sp_ref[1])
        y = jnp.maximum(t, 0.01 * t)
        out02_ref[...] = jnp.dot(
            y, W0_ref[...], preferred_element_type=jnp.float32)
        h_ref[...] = dinv * jnp.dot(
            y, W1_ref[...], preferred_element_type=jnp.float32)

    return pl.pallas_call(
        body,
        grid=(GRID,),
        in_specs=[
            pl.BlockSpec((RB, D), lambda i: (i, 0)),
            pl.BlockSpec((NC, RB, D), lambda i: (0, i, 0)),
            pl.BlockSpec((RB, 1), lambda i: (i, 0)),
            pl.BlockSpec((D, D), lambda i: (0, 0)),
            pl.BlockSpec((D, D), lambda i: (0, 0)),
        ],
        out_specs=[
            pl.BlockSpec((RB, D), lambda i: (i, 0)),
            pl.BlockSpec((RB, D), lambda i: (i, 0)),
        ],
        out_shape=[
            jax.ShapeDtypeStruct((N, D), jnp.float32),
            jax.ShapeDtypeStruct((N, D), jnp.float32),
        ],
    )(out0, sp, dinv, W0, W1)


def _tc_final(out02, sp, dinv):
    def body(o_ref, sp_ref, dinv_ref, out_ref):
        t = o_ref[...] + dinv_ref[...] * (sp_ref[0] + sp_ref[1])
        out_ref[...] = jnp.maximum(t, 0.01 * t)

    return pl.pallas_call(
        body,
        grid=(GRID,),
        in_specs=[
            pl.BlockSpec((RB, D), lambda i: (i, 0)),
            pl.BlockSpec((NC, RB, D), lambda i: (0, i, 0)),
            pl.BlockSpec((RB, 1), lambda i: (i, 0)),
        ],
        out_specs=pl.BlockSpec((RB, D), lambda i: (i, 0)),
        out_shape=jax.ShapeDtypeStruct((N, D), jnp.float32),
    )(out02, sp, dinv)


def kernel(x, edge_index, W0_0, W1_0, b0, W0_1, W1_1):
    row = edge_index[0]
    col = edge_index[1]
    degp = _sc_degree(col).reshape(NC, N, 1)
    out0, h1, dinv = _tc_layer1(x, W0_0, W1_0, b0.reshape(1, D), degp)
    s1 = _sc_scatter(h1, row, col)
    out02, h2 = _tc_layer2(out0, s1, dinv, W0_1, W1_1)
    s2 = _sc_scatter(h2, row, col)
    return _tc_final(out02, s2, dinv)
